# baseline (speedup 1.0000x reference)
import functools

import jax
import jax.numpy as jnp
from jax import lax
from jax.experimental import pallas as pl
from jax.experimental.pallas import tpu as pltpu
from jax.experimental.pallas import tpu_sc as plsc

NUM_EXPERTS = 64
TOP_K = 2
BATCH = 4
SEQ = 8192
TOKENS = BATCH * SEQ

_INFO = plsc.get_sparse_core_info()
NC = _INFO.num_cores
NS = _INFO.num_subcores
LANES = _INFO.num_lanes
NW = NC * NS
TPW = TOKENS // NW
WPB = SEQ // TPW
CHUNK = 256
GPC = CHUNK // LANES
UNROLL = 16

_mesh = plsc.VectorSubcoreMesh(core_axis_name="c", subcore_axis_name="s")


@functools.partial(
    pl.kernel,
    out_type=(
        jax.ShapeDtypeStruct((TOP_K, BATCH, SEQ), jnp.int32),
        jax.ShapeDtypeStruct((TOP_K, BATCH, SEQ), jnp.float32),
        jax.ShapeDtypeStruct((TOP_K, BATCH, SEQ), jnp.float32),
    ),
    mesh=_mesh,
    compiler_params=pltpu.CompilerParams(needs_layout_passes=False),
    scratch_types=[
        pltpu.VMEM((CHUNK, NUM_EXPERTS), jnp.float32),
        pltpu.VMEM((CHUNK, NUM_EXPERTS), jnp.float32),
        pltpu.SemaphoreType.DMA,
        pltpu.SemaphoreType.DMA,
        pltpu.VMEM((NUM_EXPERTS,), jnp.float32),
        pltpu.VMEM((TPW,), jnp.int32),
        pltpu.VMEM((TPW,), jnp.int32),
        pltpu.VMEM((TPW,), jnp.float32),
        pltpu.VMEM((TPW,), jnp.float32),
        pltpu.VMEM((TPW,), jnp.float32),
    ],
)
def _auction(conf_hbm, wealth_hbm, oidx_hbm, orw_hbm, opay_hbm,
             conf_a, conf_b, sem_a, sem_b, wealth_v, e0_v, e1_v, w0_v,
             w1_v, p_v):
    wid = lax.axis_index("s") * NC + lax.axis_index("c")
    b = wid // WPB
    row0 = (wid % WPB) * TPW
    pltpu.sync_copy(wealth_hbm, wealth_v)

    iota = lax.iota(jnp.int32, LANES)
    zeros = jnp.zeros((LANES,), jnp.int32)
    neg_inf = jnp.full((LANES,), -jnp.inf, jnp.float32)

    bufs = (conf_a, conf_b)
    sems = (sem_a, sem_b)
    nchunk = TPW // CHUNK

    def start_fetch(c):
        return pltpu.async_copy(
            conf_hbm.at[b, pl.ds(row0 + c * CHUNK, CHUNK)],
            bufs[c % 2], sems[c % 2])

    handles = [start_fetch(0), None]
    for chunk in range(nchunk):
        if chunk + 1 < nchunk:
            handles[(chunk + 1) % 2] = start_fetch(chunk + 1)
        handles[chunk % 2].wait()
        conf_v = bufs[chunk % 2]

        def group_body(g, carry):
            uv = iota + g * LANES

            def expert_body(_, st):
                m1, m2, m3, i1, i2, ev = st
                for k in range(UNROLL):
                    evk = ev + k
                    col = plsc.load_gather(conf_v, [uv, evk])
                    w = plsc.load_gather(wealth_v, [evk])
                    b_ = col * w
                    gt1 = b_ > m1
                    gt2 = b_ > m2
                    nm3 = jnp.maximum(m3, jnp.minimum(m2, b_))
                    nm2 = jnp.maximum(m2, jnp.minimum(m1, b_))
                    ni2 = jnp.where(gt1, i1, jnp.where(gt2, evk, i2))
                    nm1 = jnp.maximum(m1, b_)
                    ni1 = jnp.where(gt1, evk, i1)
                    m1, m2, m3, i1, i2 = nm1, nm2, nm3, ni1, ni2
                return m1, m2, m3, i1, i2, ev + UNROLL

            m1, m2, m3, i1, i2, _ = lax.fori_loop(
                0, NUM_EXPERTS // UNROLL, expert_body,
                (neg_inf, neg_inf, neg_inf, zeros, zeros, zeros))

            t = jnp.exp(m2 - m1)
            inv = 1.0 / (1.0 + t)
            off = chunk * CHUNK + g * LANES
            e0_v[pl.ds(off, LANES)] = i1
            e1_v[pl.ds(off, LANES)] = i2
            w0_v[pl.ds(off, LANES)] = inv
            w1_v[pl.ds(off, LANES)] = t * inv
            p_v[pl.ds(off, LANES)] = m3
            return carry

        lax.fori_loop(0, GPC, group_body, 0)

    pltpu.sync_copy(e0_v, oidx_hbm.at[0, b, pl.ds(row0, TPW)])
    pltpu.sync_copy(e1_v, oidx_hbm.at[1, b, pl.ds(row0, TPW)])
    pltpu.sync_copy(w0_v, orw_hbm.at[0, b, pl.ds(row0, TPW)])
    pltpu.sync_copy(w1_v, orw_hbm.at[1, b, pl.ds(row0, TPW)])
    pltpu.sync_copy(p_v, opay_hbm.at[0, b, pl.ds(row0, TPW)])
    pltpu.sync_copy(p_v, opay_hbm.at[1, b, pl.ds(row0, TPW)])


def kernel(confidences, wealth):
    oidx, orw, opay = _auction(confidences, wealth)
    perm = (1, 2, 0)
    return (jnp.transpose(oidx, perm), jnp.transpose(orw, perm),
            jnp.transpose(opay, perm))


# speedup vs baseline: 1.0001x; 1.0001x over previous
import functools

import jax
import jax.numpy as jnp
from jax import lax
from jax.experimental import pallas as pl
from jax.experimental.pallas import tpu as pltpu
from jax.experimental.pallas import tpu_sc as plsc

NUM_EXPERTS = 64
TOP_K = 2
BATCH = 4
SEQ = 8192
TOKENS = BATCH * SEQ

_INFO = plsc.get_sparse_core_info()
NC = _INFO.num_cores
NS = _INFO.num_subcores
LANES = _INFO.num_lanes
NW = NC * NS
TPW = TOKENS // NW
WPB = SEQ // TPW
CHUNK = 256
GPC = CHUNK // LANES
UNROLL = 4

_mesh = plsc.VectorSubcoreMesh(core_axis_name="c", subcore_axis_name="s")


@functools.partial(
    pl.kernel,
    out_type=(
        jax.ShapeDtypeStruct((TOP_K, BATCH, SEQ), jnp.int32),
        jax.ShapeDtypeStruct((TOP_K, BATCH, SEQ), jnp.float32),
        jax.ShapeDtypeStruct((TOP_K, BATCH, SEQ), jnp.float32),
    ),
    mesh=_mesh,
    compiler_params=pltpu.CompilerParams(needs_layout_passes=False),
    scratch_types=[
        pltpu.VMEM((CHUNK, NUM_EXPERTS), jnp.float32),
        pltpu.VMEM((CHUNK, NUM_EXPERTS), jnp.float32),
        pltpu.SemaphoreType.DMA,
        pltpu.SemaphoreType.DMA,
        pltpu.VMEM((NUM_EXPERTS,), jnp.float32),
        pltpu.VMEM((TPW,), jnp.int32),
        pltpu.VMEM((TPW,), jnp.int32),
        pltpu.VMEM((TPW,), jnp.float32),
        pltpu.VMEM((TPW,), jnp.float32),
        pltpu.VMEM((TPW,), jnp.float32),
    ],
)
def _auction(conf_hbm, wealth_hbm, oidx_hbm, orw_hbm, opay_hbm,
             conf_a, conf_b, sem_a, sem_b, wealth_v, e0_v, e1_v, w0_v,
             w1_v, p_v):
    wid = lax.axis_index("s") * NC + lax.axis_index("c")
    b = wid // WPB
    row0 = (wid % WPB) * TPW
    pltpu.sync_copy(wealth_hbm, wealth_v)

    iota = lax.iota(jnp.int32, LANES)
    zeros = jnp.zeros((LANES,), jnp.int32)
    neg_inf = jnp.full((LANES,), -jnp.inf, jnp.float32)

    bufs = (conf_a, conf_b)
    sems = (sem_a, sem_b)
    nchunk = TPW // CHUNK

    def start_fetch(c):
        return pltpu.async_copy(
            conf_hbm.at[b, pl.ds(row0 + c * CHUNK, CHUNK)],
            bufs[c % 2], sems[c % 2])

    handles = [start_fetch(0), None]
    for chunk in range(nchunk):
        if chunk + 1 < nchunk:
            handles[(chunk + 1) % 2] = start_fetch(chunk + 1)
        handles[chunk % 2].wait()
        conf_v = bufs[chunk % 2]

        def group_body(g, carry):
            uva = iota + g * (2 * LANES)
            uvb = uva + LANES

            def expert_body(_, st):
                sa, sb, ev = st
                for k in range(UNROLL):
                    evk = ev + k
                    w = plsc.load_gather(wealth_v, [evk])
                    ns = []
                    for uv, (m1, m2, m3, i1, i2) in ((uva, sa), (uvb, sb)):
                        col = plsc.load_gather(conf_v, [uv, evk])
                        b_ = col * w
                        gt1 = b_ > m1
                        gt2 = b_ > m2
                        nm3 = jnp.maximum(m3, jnp.minimum(m2, b_))
                        nm2 = jnp.maximum(m2, jnp.minimum(m1, b_))
                        ni2 = jnp.where(gt1, i1, jnp.where(gt2, evk, i2))
                        nm1 = jnp.maximum(m1, b_)
                        ni1 = jnp.where(gt1, evk, i1)
                        ns.append((nm1, nm2, nm3, ni1, ni2))
                    sa, sb = ns
                return sa, sb, ev + UNROLL

            init = (neg_inf, neg_inf, neg_inf, zeros, zeros)
            sa, sb, _ = lax.fori_loop(
                0, NUM_EXPERTS // UNROLL, expert_body, (init, init, zeros))

            for half, (m1, m2, m3, i1, i2) in enumerate((sa, sb)):
                t = jnp.exp(m2 - m1)
                inv = 1.0 / (1.0 + t)
                off = chunk * CHUNK + g * (2 * LANES) + half * LANES
                e0_v[pl.ds(off, LANES)] = i1
                e1_v[pl.ds(off, LANES)] = i2
                w0_v[pl.ds(off, LANES)] = inv
                w1_v[pl.ds(off, LANES)] = t * inv
                p_v[pl.ds(off, LANES)] = m3
            return carry

        lax.fori_loop(0, GPC // 2, group_body, 0)

    pltpu.sync_copy(e0_v, oidx_hbm.at[0, b, pl.ds(row0, TPW)])
    pltpu.sync_copy(e1_v, oidx_hbm.at[1, b, pl.ds(row0, TPW)])
    pltpu.sync_copy(w0_v, orw_hbm.at[0, b, pl.ds(row0, TPW)])
    pltpu.sync_copy(w1_v, orw_hbm.at[1, b, pl.ds(row0, TPW)])
    pltpu.sync_copy(p_v, opay_hbm.at[0, b, pl.ds(row0, TPW)])
    pltpu.sync_copy(p_v, opay_hbm.at[1, b, pl.ds(row0, TPW)])


def kernel(confidences, wealth):
    oidx, orw, opay = _auction(confidences, wealth)
    perm = (1, 2, 0)
    return (jnp.transpose(oidx, perm), jnp.transpose(orw, perm),
            jnp.transpose(opay, perm))


# speedup vs baseline: 1.0115x; 1.0114x over previous
import functools

import jax
import jax.numpy as jnp
from jax import lax
from jax.experimental import pallas as pl
from jax.experimental.pallas import tpu as pltpu
from jax.experimental.pallas import tpu_sc as plsc

NUM_EXPERTS = 64
TOP_K = 2
BATCH = 4
SEQ = 8192
TOKENS = BATCH * SEQ

_INFO = plsc.get_sparse_core_info()
NC = _INFO.num_cores
NS = _INFO.num_subcores
LANES = _INFO.num_lanes
NW = NC * NS
TPW = TOKENS // NW
WPB = SEQ // TPW
CHUNK = 256
GPC = CHUNK // LANES
UNROLL = 8

_mesh = plsc.VectorSubcoreMesh(core_axis_name="c", subcore_axis_name="s")


@functools.partial(
    pl.kernel,
    out_type=(
        jax.ShapeDtypeStruct((TOP_K, BATCH, SEQ), jnp.int32),
        jax.ShapeDtypeStruct((TOP_K, BATCH, SEQ), jnp.float32),
        jax.ShapeDtypeStruct((TOP_K, BATCH, SEQ), jnp.float32),
    ),
    mesh=_mesh,
    compiler_params=pltpu.CompilerParams(needs_layout_passes=False),
    scratch_types=[
        pltpu.VMEM((CHUNK, NUM_EXPERTS), jnp.float32),
        pltpu.VMEM((CHUNK, NUM_EXPERTS), jnp.float32),
        pltpu.SemaphoreType.DMA,
        pltpu.SemaphoreType.DMA,
        pltpu.VMEM((NUM_EXPERTS,), jnp.float32),
        pltpu.VMEM((TPW,), jnp.int32),
        pltpu.VMEM((TPW,), jnp.int32),
        pltpu.VMEM((TPW,), jnp.float32),
        pltpu.VMEM((TPW,), jnp.float32),
        pltpu.VMEM((TPW,), jnp.float32),
    ],
)
def _auction(conf_hbm, wealth_hbm, oidx_hbm, orw_hbm, opay_hbm,
             conf_a, conf_b, sem_a, sem_b, wealth_v, e0_v, e1_v, w0_v,
             w1_v, p_v):
    wid = lax.axis_index("s") * NC + lax.axis_index("c")
    b = wid // WPB
    row0 = (wid % WPB) * TPW
    pltpu.sync_copy(wealth_hbm, wealth_v)

    iota = lax.iota(jnp.int32, LANES)
    zeros = jnp.zeros((LANES,), jnp.int32)
    neg_inf = jnp.full((LANES,), -jnp.inf, jnp.float32)

    bufs = (conf_a, conf_b)
    sems = (sem_a, sem_b)
    nchunk = TPW // CHUNK

    def start_fetch(c):
        return pltpu.async_copy(
            conf_hbm.at[b, pl.ds(row0 + c * CHUNK, CHUNK)],
            bufs[c % 2], sems[c % 2])

    handles = [start_fetch(0), None]
    for chunk in range(nchunk):
        if chunk + 1 < nchunk:
            handles[(chunk + 1) % 2] = start_fetch(chunk + 1)
        handles[chunk % 2].wait()
        conf_v = bufs[chunk % 2]

        def group_body(g, carry):
            uv = iota + g * LANES

            def expert_body(_, st):
                m1, m2, m3, i1, i2, ev = st
                for k in range(UNROLL):
                    evk = ev + k
                    col = plsc.load_gather(conf_v, [uv, evk])
                    w = plsc.load_gather(wealth_v, [evk])
                    b_ = col * w
                    gt1 = b_ > m1
                    gt2 = b_ > m2
                    nm3 = jnp.maximum(m3, jnp.minimum(m2, b_))
                    nm2 = jnp.maximum(m2, jnp.minimum(m1, b_))
                    ni2 = jnp.where(gt1, i1, jnp.where(gt2, evk, i2))
                    nm1 = jnp.maximum(m1, b_)
                    ni1 = jnp.where(gt1, evk, i1)
                    m1, m2, m3, i1, i2 = nm1, nm2, nm3, ni1, ni2
                return m1, m2, m3, i1, i2, ev + UNROLL

            m1, m2, m3, i1, i2, _ = lax.fori_loop(
                0, NUM_EXPERTS // UNROLL, expert_body,
                (neg_inf, neg_inf, neg_inf, zeros, zeros, zeros))

            t = jnp.exp(m2 - m1)
            inv = 1.0 / (1.0 + t)
            off = chunk * CHUNK + g * LANES
            e0_v[pl.ds(off, LANES)] = i1
            e1_v[pl.ds(off, LANES)] = i2
            w0_v[pl.ds(off, LANES)] = inv
            w1_v[pl.ds(off, LANES)] = t * inv
            p_v[pl.ds(off, LANES)] = m3
            return carry

        lax.fori_loop(0, GPC, group_body, 0)

    pltpu.sync_copy(e0_v, oidx_hbm.at[0, b, pl.ds(row0, TPW)])
    pltpu.sync_copy(e1_v, oidx_hbm.at[1, b, pl.ds(row0, TPW)])
    pltpu.sync_copy(w0_v, orw_hbm.at[0, b, pl.ds(row0, TPW)])
    pltpu.sync_copy(w1_v, orw_hbm.at[1, b, pl.ds(row0, TPW)])
    pltpu.sync_copy(p_v, opay_hbm.at[0, b, pl.ds(row0, TPW)])
    pltpu.sync_copy(p_v, opay_hbm.at[1, b, pl.ds(row0, TPW)])


def kernel(confidences, wealth):
    oidx, orw, opay = _auction(confidences, wealth)
    perm = (1, 2, 0)
    return (jnp.transpose(oidx, perm), jnp.transpose(orw, perm),
            jnp.transpose(opay, perm))
